# baseline (speedup 1.0000x reference)
import functools

import jax
import jax.numpy as jnp
from jax import lax
from jax.experimental import pallas as pl
from jax.experimental.pallas import tpu as pltpu
from jax.experimental.pallas import tpu_sc as plsc

N = 10000
E = 320000
D = 128
NC = 2
NS = 16
NW = NC * NS
EPW = E // NW
C = 128
CPT = 79
SLOTS = CPT * C
NP = 10240
RPT = NP // NS
ZR = 80


def _enc_body(x_ref, w1_ref, b1_ref, w2_ref, b2_ref, o_ref):
    h = jnp.dot(x_ref[...], w1_ref[...], preferred_element_type=jnp.float32)
    h = jnp.maximum(h + b1_ref[...], 0.0)
    h = jnp.dot(h, w2_ref[...], preferred_element_type=jnp.float32)
    o_ref[...] = jnp.maximum(h + b2_ref[...], 0.0)


def _dec_body(p_ref, w3_ref, b3_ref, w4_ref, b4_ref, o_ref):
    agg = p_ref[0] + p_ref[1]
    h = jnp.dot(agg, w3_ref[...], preferred_element_type=jnp.float32)
    h = jnp.maximum(h + b3_ref[...], 0.0)
    h = jnp.dot(h, w4_ref[...], preferred_element_type=jnp.float32)
    o_ref[...] = jnp.maximum(h + b4_ref[...], 0.0)


def _sc_body(h_hbm, meta_hbm, out_hbm,
             meta0, rows0, dstS0, meta1, rows1, dstS1,
             zero_v, agg_sh, sg0, ss0, sc0, sg1, ss1, sc1):
    c = lax.axis_index("c")
    s = lax.axis_index("s")
    wid = s * NC + c
    cbase = wid * CPT
    bufs = ((meta0, rows0, dstS0, sg0, ss0, sc0),
            (meta1, rows1, dstS1, sg1, ss1, sc1))

    def zrow(i, _):
        for j in range(D // 16):
            zero_v[i, pl.ds(j * 16, 16)] = jnp.zeros((16,), jnp.float32)
        return _
    lax.fori_loop(0, ZR, zrow, None)
    for k in range(RPT // ZR):
        pltpu.sync_copy(zero_v, agg_sh.at[pl.ds(s * RPT + k * ZR, ZR)])
    plsc.subcore_barrier()

    def prefetch(t, b):
        pltpu.async_copy(meta_hbm.at[cbase + t], b[0], b[4])

    def wait_meta(b):
        pltpu.make_async_copy(meta_hbm.at[0], b[0], b[4]).wait()

    def issue_gather(b):
        pltpu.async_copy(h_hbm.at[b[0].at[0]], b[1], b[3])

    def wait_gather(b):
        pltpu.make_async_copy(h_hbm.at[pl.ds(0, C)], b[1], b[3]).wait()

    def copy_dst(b):
        for q in range(C // 16):
            sl = pl.ds(q * 16, 16)
            b[2][sl] = b[0][1, sl]

    def sync_scatter(b):
        pltpu.sync_copy(b[1], agg_sh.at[b[2]], add=True)

    def scale(b):
        @plsc.parallel_loop(0, C, step=16)
        def grp(g):
            vg = lax.bitcast_convert_type(b[0][2, pl.ds(g, 16)], jnp.float32)
            for l in range(16):
                nrm = jnp.broadcast_to(vg[l], (16,))
                for j in range(D // 16):
                    sl = pl.ds(j * 16, 16)
                    b[1][g + l, sl] = b[1][g + l, sl] * nrm

    prefetch(0, bufs[0])
    wait_meta(bufs[0])
    issue_gather(bufs[0])
    prefetch(1, bufs[1])
    wait_gather(bufs[0])
    wait_meta(bufs[1])
    issue_gather(bufs[1])
    copy_dst(bufs[0])
    scale(bufs[0])
    sync_scatter(bufs[0])
    prefetch(2, bufs[0])

    def pair(p, _):
        t2 = 2 * p + 2
        b0, b1 = bufs[0], bufs[1]
        wait_gather(b1)
        wait_meta(b0)
        issue_gather(b0)
        copy_dst(b1)
        scale(b1)
        d1 = pltpu.async_copy(b1[1], agg_sh.at[b1[2]], b1[5], add=True)

        @pl.when(t2 + 1 < CPT)
        def _():
            prefetch(t2 + 1, b1)
        wait_gather(b0)
        d1.wait()

        @pl.when(t2 + 1 < CPT)
        def _():
            wait_meta(b1)
            issue_gather(b1)
        copy_dst(b0)
        scale(b0)
        sync_scatter(b0)

        @pl.when(t2 + 2 < CPT)
        def _():
            prefetch(t2 + 2, b0)
        return _
    lax.fori_loop(0, (CPT - 1) // 2, pair, None)
    plsc.subcore_barrier()

    pltpu.sync_copy(agg_sh.at[pl.ds(s * RPT, RPT)],
                    out_hbm.at[c, pl.ds(s * RPT, RPT)])


def _pack_meta(src, dst, norm):
    w = jnp.arange(NW, dtype=jnp.int32)[:, None]
    slot = jnp.arange(SLOTS, dtype=jnp.int32)[None, :]
    eidx = w * EPW + slot
    valid = slot < EPW
    esafe = jnp.where(valid, eidx, 0).reshape(-1)
    spread = (w * 331 + slot) % N
    src_t = jnp.where(valid, src[esafe].reshape(NW, SLOTS), spread)
    dst_t = jnp.where(valid, dst[esafe].reshape(NW, SLOTS), spread)
    nrm_t = jnp.where(valid, norm[esafe].reshape(NW, SLOTS), 0.0)
    meta = jnp.stack(
        [src_t.reshape(NW, CPT, C),
         dst_t.reshape(NW, CPT, C),
         jax.lax.bitcast_convert_type(nrm_t, jnp.int32).reshape(NW, CPT, C)],
        axis=2)
    return meta.reshape(NW * CPT, 3, C)


def _sc_aggregate(h, src, dst, norm):
    meta = _pack_meta(src, dst, norm)
    mesh = plsc.VectorSubcoreMesh(core_axis_name="c", subcore_axis_name="s")
    return pl.kernel(
        _sc_body,
        out_type=jax.ShapeDtypeStruct((NC, NP, D), jnp.float32),
        mesh=mesh,
        scratch_types=(
            [pltpu.VMEM((3, C), jnp.int32),
             pltpu.VMEM((C, D), jnp.float32),
             pltpu.VMEM((C,), jnp.int32)] * 2
            + [pltpu.VMEM((ZR, D), jnp.float32),
               pltpu.VMEM_SHARED((NP, D), jnp.float32)]
            + [pltpu.SemaphoreType.DMA] * 6
        ),
    )(h, meta)


def _mlp(body, xs, w_a, b_a, w_b, b_b, rows_blk, n_out):
    grid = n_out // rows_blk
    if xs.ndim == 3:
        x_spec = pl.BlockSpec((xs.shape[0], rows_blk, D), lambda i: (0, i, 0))
    else:
        x_spec = pl.BlockSpec((rows_blk, D), lambda i: (i, 0))
    full = lambda shape: pl.BlockSpec(shape, lambda i: tuple(0 for _ in shape))
    return pl.pallas_call(
        body,
        grid=(grid,),
        in_specs=[
            x_spec,
            full(w_a.shape), full(b_a.shape),
            full(w_b.shape), full(b_b.shape),
        ],
        out_specs=pl.BlockSpec((rows_blk, D), lambda i: (i, 0)),
        out_shape=jax.ShapeDtypeStruct((n_out, D), jnp.float32),
    )(xs, w_a, b_a, w_b, b_b)


def kernel(x, edge_index, norm, W1, b1, W2, b2, W3, b3, W4, b4):
    h = _mlp(_enc_body, x, W1.T, b1.reshape(1, D), W2.T, b2.reshape(1, D),
             rows_blk=1000, n_out=N)
    parts = _sc_aggregate(h, edge_index[0], edge_index[1], norm)
    o = _mlp(_dec_body, parts, W3.T, b3.reshape(1, D), W4.T, b4.reshape(1, D),
             rows_blk=1000, n_out=N)
    return o


# speedup vs baseline: 1.4964x; 1.4964x over previous
import functools

import jax
import jax.numpy as jnp
from jax import lax
from jax.experimental import pallas as pl
from jax.experimental.pallas import tpu as pltpu
from jax.experimental.pallas import tpu_sc as plsc

N = 10000
E = 320000
D = 128
NC = 2
NS = 16
NW = NC * NS
EPW = E // NW
C = 128
CPT = 79
SLOTS = CPT * C
NP = 10240
RPT = NP // NS
ZR = 80


def _enc_body(x_ref, w1_ref, b1_ref, w2_ref, b2_ref, o_ref):
    h = jnp.dot(x_ref[...], w1_ref[...], preferred_element_type=jnp.float32)
    h = jnp.maximum(h + b1_ref[...], 0.0)
    h = jnp.dot(h, w2_ref[...], preferred_element_type=jnp.float32)
    o_ref[...] = jnp.maximum(h + b2_ref[...], 0.0)


def _dec_body(p_ref, w3_ref, b3_ref, w4_ref, b4_ref, o_ref):
    agg = p_ref[0] + p_ref[1]
    h = jnp.dot(agg, w3_ref[...], preferred_element_type=jnp.float32)
    h = jnp.maximum(h + b3_ref[...], 0.0)
    h = jnp.dot(h, w4_ref[...], preferred_element_type=jnp.float32)
    o_ref[...] = jnp.maximum(h + b4_ref[...], 0.0)


def _sc_body(h_hbm, meta_hbm, out_hbm,
             meta0, rows0, dstS0, meta1, rows1, dstS1,
             zero_v, agg_sh, sg0, ss0, sc0, sg1, ss1, sc1):
    c = lax.axis_index("c")
    s = lax.axis_index("s")
    wid = s * NC + c
    cbase = wid * CPT
    bufs = ((meta0, rows0, dstS0, sg0, ss0, sc0),
            (meta1, rows1, dstS1, sg1, ss1, sc1))

    def zrow(i, _):
        for j in range(D // 16):
            zero_v[i, pl.ds(j * 16, 16)] = jnp.zeros((16,), jnp.float32)
        return _
    lax.fori_loop(0, ZR, zrow, None)
    for k in range(RPT // ZR):
        pltpu.sync_copy(zero_v, agg_sh.at[pl.ds(s * RPT + k * ZR, ZR)])
    plsc.subcore_barrier()

    def prefetch(t, b):
        pltpu.async_copy(meta_hbm.at[pl.ds((cbase + t) * 3 * C, 3 * C)],
                         b[0], b[4])

    def wait_meta(b):
        pltpu.make_async_copy(meta_hbm.at[pl.ds(0, 3 * C)], b[0], b[4]).wait()

    def issue_gather(b):
        pltpu.async_copy(h_hbm.at[b[0].at[pl.ds(0, C)]], b[1], b[3])

    def wait_gather(b):
        pltpu.make_async_copy(h_hbm.at[pl.ds(0, C)], b[1], b[3]).wait()

    def copy_dst(b):
        for q in range(C // 16):
            b[2][pl.ds(q * 16, 16)] = b[0][pl.ds(C + q * 16, 16)]

    def sync_scatter(b):
        pltpu.sync_copy(b[1], agg_sh.at[b[2]], add=True)

    def scale(b):
        @plsc.parallel_loop(0, C, step=16)
        def grp(g):
            vg = lax.bitcast_convert_type(b[0][pl.ds(2 * C + g, 16)],
                                          jnp.float32)
            for l in range(16):
                nrm = jnp.broadcast_to(vg[l], (16,))
                for j in range(D // 16):
                    sl = pl.ds(j * 16, 16)
                    b[1][g + l, sl] = b[1][g + l, sl] * nrm

    prefetch(0, bufs[0])
    wait_meta(bufs[0])
    issue_gather(bufs[0])
    prefetch(1, bufs[1])
    wait_gather(bufs[0])
    wait_meta(bufs[1])
    issue_gather(bufs[1])
    copy_dst(bufs[0])
    scale(bufs[0])
    sync_scatter(bufs[0])
    prefetch(2, bufs[0])

    def pair(p, _):
        t2 = 2 * p + 2
        b0, b1 = bufs[0], bufs[1]
        wait_gather(b1)
        wait_meta(b0)
        issue_gather(b0)
        copy_dst(b1)
        scale(b1)
        d1 = pltpu.async_copy(b1[1], agg_sh.at[b1[2]], b1[5], add=True)

        @pl.when(t2 + 1 < CPT)
        def _():
            prefetch(t2 + 1, b1)
        wait_gather(b0)
        d1.wait()

        @pl.when(t2 + 1 < CPT)
        def _():
            wait_meta(b1)
            issue_gather(b1)
        copy_dst(b0)
        scale(b0)
        sync_scatter(b0)

        @pl.when(t2 + 2 < CPT)
        def _():
            prefetch(t2 + 2, b0)
        return _
    lax.fori_loop(0, (CPT - 1) // 2, pair, None)
    plsc.subcore_barrier()

    pltpu.sync_copy(agg_sh.at[pl.ds(s * RPT, RPT)],
                    out_hbm.at[c, pl.ds(s * RPT, RPT)])


def _pack_meta(src, dst, norm):
    pad = (jnp.arange(NW, dtype=jnp.int32)[:, None] * 331
           + jnp.arange(SLOTS - EPW, dtype=jnp.int32)[None, :]) % N
    src_t = jnp.concatenate([src.reshape(NW, EPW), pad], axis=1)
    dst_t = jnp.concatenate([dst.reshape(NW, EPW), pad], axis=1)
    nrm_t = jnp.concatenate(
        [norm.reshape(NW, EPW),
         jnp.zeros((NW, SLOTS - EPW), jnp.float32)], axis=1)
    meta = jnp.stack(
        [src_t.reshape(NW, CPT, C),
         dst_t.reshape(NW, CPT, C),
         jax.lax.bitcast_convert_type(nrm_t, jnp.int32).reshape(NW, CPT, C)],
        axis=2)
    return meta.reshape(NW * CPT * 3 * C)


def _sc_aggregate(h, src, dst, norm):
    meta = _pack_meta(src, dst, norm)
    mesh = plsc.VectorSubcoreMesh(core_axis_name="c", subcore_axis_name="s")
    return pl.kernel(
        _sc_body,
        out_type=jax.ShapeDtypeStruct((NC, NP, D), jnp.float32),
        mesh=mesh,
        scratch_types=(
            [pltpu.VMEM((3 * C,), jnp.int32),
             pltpu.VMEM((C, D), jnp.float32),
             pltpu.VMEM((C,), jnp.int32)] * 2
            + [pltpu.VMEM((ZR, D), jnp.float32),
               pltpu.VMEM_SHARED((NP, D), jnp.float32)]
            + [pltpu.SemaphoreType.DMA] * 6
        ),
    )(h, meta)


def _mlp(body, xs, w_a, b_a, w_b, b_b, rows_blk, n_out):
    grid = n_out // rows_blk
    if xs.ndim == 3:
        x_spec = pl.BlockSpec((xs.shape[0], rows_blk, D), lambda i: (0, i, 0))
    else:
        x_spec = pl.BlockSpec((rows_blk, D), lambda i: (i, 0))
    full = lambda shape: pl.BlockSpec(shape, lambda i: tuple(0 for _ in shape))
    return pl.pallas_call(
        body,
        grid=(grid,),
        in_specs=[
            x_spec,
            full(w_a.shape), full(b_a.shape),
            full(w_b.shape), full(b_b.shape),
        ],
        out_specs=pl.BlockSpec((rows_blk, D), lambda i: (i, 0)),
        out_shape=jax.ShapeDtypeStruct((n_out, D), jnp.float32),
    )(xs, w_a, b_a, w_b, b_b)


def kernel(x, edge_index, norm, W1, b1, W2, b2, W3, b3, W4, b4):
    h = _mlp(_enc_body, x, W1.T, b1.reshape(1, D), W2.T, b2.reshape(1, D),
             rows_blk=1000, n_out=N)
    parts = _sc_aggregate(h, edge_index[0], edge_index[1], norm)
    o = _mlp(_dec_body, parts, W3.T, b3.reshape(1, D), W4.T, b4.reshape(1, D),
             rows_blk=1000, n_out=N)
    return o
